# baseline (speedup 1.0000x reference)
import functools

import jax
import jax.numpy as jnp
from jax import lax
from jax.experimental import pallas as pl
from jax.experimental.pallas import tpu as pltpu
from jax.experimental.pallas import tpu_sc as plsc

NUM_FIELDS = 26
VOCAB = 100000
EMB = 32
BATCH = 16384

NC = 2
NS = 16
NW = NC * NS
ROWS_PER_W = BATCH // NW
CHUNK = 16
N_CHUNKS = ROWS_PER_W // CHUNK
LANES = 16
ROW_W = 128

NFG = NUM_FIELDS // 2
IDXG_PER_W = NFG * ROWS_PER_W

TCOLS = 32768
LOG_TCOLS = 15
TBLK = TCOLS // 4
LOG_TBLK = 13
NCB = -(-VOCAB // TCOLS)
LPF = NCB * TBLK


def _tc_repack():
    def body(tabt_ref, out_ref):
        blk = tabt_ref[0]
        stacked = jnp.concatenate(
            [blk[:, a * TBLK:(a + 1) * TBLK] for a in range(4)], axis=0
        )
        out_ref[0] = lax.dot_general(
            stacked,
            jnp.eye(ROW_W, dtype=jnp.float32),
            (((0,), (0,)), ((), ())),
        )

    return pl.pallas_call(
        body,
        grid=(NFG, NCB),
        in_specs=[
            pl.BlockSpec((1, EMB, TCOLS), lambda f, c: (f, 0, c)),
        ],
        out_specs=pl.BlockSpec((1, TBLK, ROW_W), lambda f, c: (f, c, 0)),
        out_shape=jax.ShapeDtypeStruct((NFG, LPF, ROW_W), jnp.float32),
    )


def _make_sc_kernel():
    mesh = plsc.VectorSubcoreMesh(core_axis_name="c", subcore_axis_name="s")

    @functools.partial(
        pl.kernel,
        out_type=jax.ShapeDtypeStruct((BATCH * EMB,), jnp.float32),
        mesh=mesh,
        compiler_params=pltpu.CompilerParams(use_tc_tiling_on_sc=True),
        scratch_types=[
            pltpu.VMEM((IDXG_PER_W,), jnp.int32),
            pltpu.VMEM((IDXG_PER_W + LANES,), jnp.int32),
            pltpu.VMEM((NFG, CHUNK, ROW_W), jnp.float32),
            pltpu.VMEM((CHUNK * EMB,), jnp.float32),
            pltpu.SemaphoreType.DMA,
        ],
    )
    def k(ftab_hbm, xf_hbm, xr_hbm, out_hbm, idxg, sub, buf, acc, sem):
        wid = lax.axis_index("s") * NC + lax.axis_index("c")

        cp1 = pltpu.async_copy(
            xf_hbm.at[pl.ds(wid * IDXG_PER_W, IDXG_PER_W)], idxg, sem
        )
        cp2 = pltpu.async_copy(
            xr_hbm.at[pl.ds(wid * IDXG_PER_W, IDXG_PER_W)],
            sub.at[pl.ds(0, IDXG_PER_W)],
            sem,
        )
        cp1.wait()
        cp2.wait()

        @pl.loop(0, IDXG_PER_W // LANES)
        def _prep(c):
            sl = pl.ds(c * LANES, LANES)
            f = (c * LANES) // ROWS_PER_W
            v = idxg[sl]
            r = v & (TCOLS - 1)
            idxg[sl] = ((v >> LOG_TCOLS) << LOG_TBLK) + (r & (TBLK - 1)) + f * LPF
            vr = sub[sl] & (TCOLS - 1)
            sub[sl] = (vr >> LOG_TBLK) << 5

        @pl.loop(0, N_CHUNKS)
        def _chunk(g):
            base = pl.multiple_of(g * CHUNK, CHUNK)
            copies = []
            for f in range(NFG):
                copies.append(
                    pltpu.async_copy(
                        ftab_hbm.at[idxg.at[pl.ds(f * ROWS_PER_W + base, CHUNK)]],
                        buf.at[f],
                        sem,
                    )
                )
            for c in copies:
                c.wait()

            @pl.loop(0, CHUNK)
            def _row(r):
                rbase = (base + r) * NFG
                sv = sub[pl.ds(rbase, LANES)]
                offs = [sv[f] for f in range(NFG)]
                for half in range(EMB // LANES):
                    t = None
                    for f in range(NFG):
                        piece = buf[f, r, pl.ds(offs[f] + half * LANES, LANES)]
                        t = piece if t is None else t + piece
                    acc[pl.ds(r * EMB + half * LANES, LANES)] = t

            pltpu.sync_copy(
                acc,
                out_hbm.at[pl.ds(wid * (ROWS_PER_W * EMB) + base * EMB, CHUNK * EMB)],
            )

    return k


_repack = _tc_repack()
_sc_kernel = _make_sc_kernel()


@jax.jit
def kernel(x, tables):
    tabt = tables.transpose(0, 2, 1)
    xi = x.astype(jnp.int32)
    partials = []
    for g in range(2):
        f0 = g * NFG
        ftab = _repack(tabt[f0:f0 + NFG]).reshape(NFG * LPF, ROW_W)
        xg = xi[:, f0:f0 + NFG]
        xf = xg.reshape(NW, ROWS_PER_W, NFG).transpose(0, 2, 1).reshape(-1)
        xr = xg.reshape(-1)
        partials.append(_sc_kernel(ftab, xf, xr))
    out = partials[0] + partials[1]
    return out.reshape(BATCH, EMB)


# speedup vs baseline: 1.3876x; 1.3876x over previous
import functools

import jax
import jax.numpy as jnp
from jax import lax
from jax.experimental import pallas as pl
from jax.experimental.pallas import tpu as pltpu
from jax.experimental.pallas import tpu_sc as plsc

NUM_FIELDS = 26
VOCAB = 100000
EMB = 32
BATCH = 16384

NC = 2
NS = 16
NW = NC * NS
ROWS_PER_W = BATCH // NW
IDX_PER_W = NUM_FIELDS * ROWS_PER_W
CHUNK = 16
N_CHUNKS = ROWS_PER_W // CHUNK
LANES = 16
ROW_W = 128
VOC4 = VOCAB // 4

TCOLS = 32768
LOG_TCOLS = 15
TBLK = TCOLS // 4
LOG_TBLK = 13
NCB = -(-VOCAB // TCOLS)
LPF = NCB * TBLK


def _tc_repack():
    def body(tabt_ref, out_ref):
        blk = tabt_ref[0]
        stacked = jnp.concatenate(
            [blk[:, a * TBLK:(a + 1) * TBLK] for a in range(4)], axis=0
        )
        out_ref[0] = lax.dot_general(
            stacked,
            jnp.eye(ROW_W, dtype=jnp.float32),
            (((0,), (0,)), ((), ())),
        )

    return pl.pallas_call(
        body,
        grid=(NUM_FIELDS, NCB),
        in_specs=[
            pl.BlockSpec((1, EMB, TCOLS), lambda f, c: (f, 0, c)),
        ],
        out_specs=pl.BlockSpec((1, TBLK, ROW_W), lambda f, c: (f, c, 0)),
        out_shape=jax.ShapeDtypeStruct((NUM_FIELDS, LPF, ROW_W), jnp.float32),
    )


def _make_sc_kernel():
    mesh = plsc.VectorSubcoreMesh(core_axis_name="c", subcore_axis_name="s")

    @functools.partial(
        pl.kernel,
        out_type=jax.ShapeDtypeStruct((BATCH * EMB,), jnp.float32),
        mesh=mesh,
        compiler_params=pltpu.CompilerParams(use_tc_tiling_on_sc=True),
        scratch_types=[
            pltpu.VMEM((IDX_PER_W,), jnp.int32),
            pltpu.VMEM((IDX_PER_W + LANES,), jnp.int32),
            pltpu.VMEM((NUM_FIELDS, CHUNK, ROW_W), jnp.float32),
            pltpu.VMEM((CHUNK * EMB,), jnp.float32),
            pltpu.SemaphoreType.DMA,
        ],
    )
    def k(ftab_hbm, xf_hbm, xr_hbm, out_hbm, idxg, sub, buf, acc, sem):
        wid = lax.axis_index("s") * NC + lax.axis_index("c")

        cp1 = pltpu.async_copy(
            xf_hbm.at[pl.ds(wid * IDX_PER_W, IDX_PER_W)], idxg, sem
        )
        cp2 = pltpu.async_copy(
            xr_hbm.at[pl.ds(wid * IDX_PER_W, IDX_PER_W)],
            sub.at[pl.ds(0, IDX_PER_W)],
            sem,
        )
        cp1.wait()
        cp2.wait()

        @pl.loop(0, IDX_PER_W // LANES)
        def _prep(c):
            sl = pl.ds(c * LANES, LANES)
            f = (c * LANES) // ROWS_PER_W
            v = idxg[sl]
            r = v & (TCOLS - 1)
            idxg[sl] = ((v >> LOG_TCOLS) << LOG_TBLK) + (r & (TBLK - 1)) + f * LPF
            vr = sub[sl] & (TCOLS - 1)
            sub[sl] = (vr >> LOG_TBLK) << 5

        @pl.loop(0, N_CHUNKS)
        def _chunk(g):
            base = pl.multiple_of(g * CHUNK, CHUNK)
            copies = []
            for f in range(NUM_FIELDS):
                copies.append(
                    pltpu.async_copy(
                        ftab_hbm.at[idxg.at[pl.ds(f * ROWS_PER_W + base, CHUNK)]],
                        buf.at[f],
                        sem,
                    )
                )
            for c in copies:
                c.wait()

            @pl.loop(0, CHUNK)
            def _row(r):
                rbase = (base + r) * NUM_FIELDS
                sv0 = sub[pl.ds(rbase, LANES)]
                sv1 = sub[pl.ds(rbase + LANES, LANES)]
                offs = [sv0[f] for f in range(LANES)] + [
                    sv1[f - LANES] for f in range(LANES, NUM_FIELDS)
                ]
                for half in range(EMB // LANES):
                    t = None
                    for f in range(NUM_FIELDS):
                        piece = buf[f, r, pl.ds(offs[f] + half * LANES, LANES)]
                        t = piece if t is None else t + piece
                    acc[pl.ds(r * EMB + half * LANES, LANES)] = t

            pltpu.sync_copy(
                acc,
                out_hbm.at[pl.ds(wid * (ROWS_PER_W * EMB) + base * EMB, CHUNK * EMB)],
            )

    return k


_repack = _tc_repack()
_sc_kernel = _make_sc_kernel()


@jax.jit
def kernel(x, tables):
    tabt = tables.transpose(0, 2, 1)
    ftab = _repack(tabt).reshape(NUM_FIELDS * LPF, ROW_W)
    xi = x.astype(jnp.int32)
    xf = xi.reshape(NW, ROWS_PER_W, NUM_FIELDS).transpose(0, 2, 1).reshape(-1)
    xr = xi.reshape(-1)
    out = _sc_kernel(ftab, xf, xr)
    return out.reshape(BATCH, EMB)
